# baseline (speedup 1.0000x reference)
import jax
import jax.numpy as jnp
from jax import lax
from jax.experimental import pallas as pl
from jax.experimental.pallas import tpu as pltpu

B, C, W, H = 8, 96, 128, 128
N = W * H
K = N // 2
G = 4
R = G * W
EPS = 1e-12


def _chunkfold(terms):
    chunks = []
    for kk in range(0, C, 32):
        acc = terms[kk]
        for c in range(kk + 1, kk + 32):
            acc = acc + terms[c]
        chunks.append(acc)
    return (chunks[0] + chunks[1]) + chunks[2]


def _dense_body(x_ref, y_ref, cos_ref, raw_ref):
    xb = x_ref[0]
    yb = y_ref[0]
    sxx = _chunkfold([xb[c] * xb[c] for c in range(C)])
    syy = _chunkfold([yb[c] * yb[c] for c in range(C)])
    nx = jnp.maximum(jnp.where(sxx == 0.0, 0.0, sxx * lax.rsqrt(sxx)), EPS)
    ny = jnp.maximum(jnp.where(syy == 0.0, 0.0, syy * lax.rsqrt(syy)), EPS)
    rx = 1.0 / nx
    ry = 1.0 / ny
    cos = _chunkfold([(rx * xb[c]) * (ry * yb[c]) for c in range(C)])
    cos_ref[0] = cos
    raw_ref[0, 0] = -cos


def _xor_partner(a, d, rowl_ids, col_ids):
    if d >= 128:
        m = d // 128
        plus = pltpu.roll(a, m, 0)
        minus = pltpu.roll(a, R - m, 0)
        bit = (rowl_ids & m) != 0
    else:
        plus = pltpu.roll(a, d, 1)
        minus = pltpu.roll(a, H - d, 1)
        bit = (col_ids & d) != 0
    return jnp.where(bit, plus, minus)


def _sort_body(cos_ref, mask_ref, idx_ref):
    cos_all = cos_ref[...].reshape(R, H)
    row_ids = lax.broadcasted_iota(jnp.int32, (R, H), 0)
    col_ids = lax.broadcasted_iota(jnp.int32, (R, H), 1)
    rowl_ids = row_ids & (W - 1)
    p_ids = rowl_ids * H + col_ids

    keys = cos_all
    idxs = p_ids
    k = 2
    while k <= N:
        j = k // 2
        while j >= 1:
            kp = _xor_partner(keys, j, rowl_ids, col_ids)
            ip = _xor_partner(idxs, j, rowl_ids, col_ids)
            lt = (keys < kp) | ((keys == kp) & (idxs < ip))
            m_lower = (p_ids & j) == 0
            asc = (p_ids & k) == 0 if k < N else jnp.full((R, H), True)
            keep = (lt == m_lower) == asc
            keys = jnp.where(keep, keys, kp)
            idxs = jnp.where(keep, idxs, ip)
            j //= 2
        k *= 2

    p2d = (lax.broadcasted_iota(jnp.int32, (W, H), 0) * H
           + lax.broadcasted_iota(jnp.int32, (W, H), 1))
    for b in range(G):
        t_cos = keys[W * b + (K - 1) // H, (K - 1) % H]
        t_idx = idxs[W * b + (K - 1) // H, (K - 1) % H]
        cb = cos_ref[b]
        mask = (cb < t_cos) | ((cb == t_cos) & (p2d <= t_idx))
        mask_ref[b, 0] = (mask.astype(jnp.float32) - cb) + cb
        idx_ref[b] = idxs[W * b: W * b + K // H, :]


def kernel(x, y):
    cos, raw = pl.pallas_call(
        _dense_body,
        grid=(B,),
        in_specs=[
            pl.BlockSpec((1, C, W, H), lambda b: (b, 0, 0, 0)),
            pl.BlockSpec((1, C, W, H), lambda b: (b, 0, 0, 0)),
        ],
        out_specs=[
            pl.BlockSpec((1, W, H), lambda b: (b, 0, 0)),
            pl.BlockSpec((1, 1, W, H), lambda b: (b, 0, 0, 0)),
        ],
        out_shape=[
            jax.ShapeDtypeStruct((B, W, H), jnp.float32),
            jax.ShapeDtypeStruct((B, 1, W, H), jnp.float32),
        ],
    )(x, y)
    mask, idx = pl.pallas_call(
        _sort_body,
        grid=(B // G,),
        in_specs=[pl.BlockSpec((G, W, H), lambda g: (g, 0, 0))],
        out_specs=[
            pl.BlockSpec((G, 1, W, H), lambda g: (g, 0, 0, 0)),
            pl.BlockSpec((G, K // H, H), lambda g: (g, 0, 0)),
        ],
        out_shape=[
            jax.ShapeDtypeStruct((B, 1, W, H), jnp.float32),
            jax.ShapeDtypeStruct((B, K // H, H), jnp.int32),
        ],
    )(cos)
    return mask, raw, idx.reshape(B, K)


# speedup vs baseline: 1.3112x; 1.3112x over previous
import jax
import jax.numpy as jnp
from jax import lax
from jax.experimental import pallas as pl
from jax.experimental.pallas import tpu as pltpu

B, C, W, H = 8, 96, 128, 128
N = W * H
K = N // 2
G = 8
R = G * W
EPS = 1e-12


def _chunkfold(terms):
    chunks = []
    for kk in range(0, C, 32):
        acc = terms[kk]
        for c in range(kk + 1, kk + 32):
            acc = acc + terms[c]
        chunks.append(acc)
    return (chunks[0] + chunks[1]) + chunks[2]


def _dense_body(x_ref, y_ref, cos_ref, raw_ref):
    xb = x_ref[0]
    yb = y_ref[0]
    sxx = _chunkfold([xb[c] * xb[c] for c in range(C)])
    syy = _chunkfold([yb[c] * yb[c] for c in range(C)])
    nx = jnp.maximum(jnp.where(sxx == 0.0, 0.0, sxx * lax.rsqrt(sxx)), EPS)
    ny = jnp.maximum(jnp.where(syy == 0.0, 0.0, syy * lax.rsqrt(syy)), EPS)
    rx = 1.0 / nx
    ry = 1.0 / ny
    cos = _chunkfold([(rx * xb[c]) * (ry * yb[c]) for c in range(C)])
    cos_ref[0] = cos
    raw_ref[0, 0] = -cos


def _xor_partner(a, d, rowl_ids, col_ids, nrows):
    if d >= 128:
        m = d // 128
        plus = pltpu.roll(a, m, 0)
        minus = pltpu.roll(a, nrows - m, 0)
        bit = (rowl_ids & m) != 0
    else:
        plus = pltpu.roll(a, d, 1)
        minus = pltpu.roll(a, H - d, 1)
        bit = (col_ids & d) != 0
    return jnp.where(bit, plus, minus)


def _sort_body(cos_ref, mask_ref, idx_ref):
    cos_all = cos_ref[...].reshape(R, H)
    row_ids = lax.broadcasted_iota(jnp.int32, (R, H), 0)
    col_ids = lax.broadcasted_iota(jnp.int32, (R, H), 1)
    rowl_ids = row_ids & (W - 1)
    p_ids = rowl_ids * H + col_ids

    keys = cos_all
    idxs = p_ids
    k = 2
    while k <= N // 2:
        asc = (p_ids & k) == 0
        j = k // 2
        while j >= 1:
            kp = _xor_partner(keys, j, rowl_ids, col_ids, R)
            ip = _xor_partner(idxs, j, rowl_ids, col_ids, R)
            lt = (keys < kp) | ((keys == kp) & (idxs < ip))
            m_lower = (p_ids & j) == 0
            keep = (lt == m_lower) == asc
            keys = jnp.where(keep, keys, kp)
            idxs = jnp.where(keep, idxs, ip)
            j //= 2
        k *= 2

    k3 = keys.reshape(G, W, H)
    i3 = idxs.reshape(G, W, H)
    W2 = W // 2
    lt = ((k3[:, :W2] < k3[:, W2:])
          | ((k3[:, :W2] == k3[:, W2:]) & (i3[:, :W2] < i3[:, W2:])))
    R2 = G * W2
    keys = jnp.where(lt, k3[:, :W2], k3[:, W2:]).reshape(R2, H)
    idxs = jnp.where(lt, i3[:, :W2], i3[:, W2:]).reshape(R2, H)
    row2 = lax.broadcasted_iota(jnp.int32, (R2, H), 0)
    col2 = lax.broadcasted_iota(jnp.int32, (R2, H), 1)
    rowl2 = row2 & (W2 - 1)
    p2 = rowl2 * H + col2
    j = N // 4
    while j >= 1:
        kp = _xor_partner(keys, j, rowl2, col2, R2)
        ip = _xor_partner(idxs, j, rowl2, col2, R2)
        lt = (keys < kp) | ((keys == kp) & (idxs < ip))
        keep = lt == ((p2 & j) == 0)
        keys = jnp.where(keep, keys, kp)
        idxs = jnp.where(keep, idxs, ip)
        j //= 2

    p2d = (lax.broadcasted_iota(jnp.int32, (W, H), 0) * H
           + lax.broadcasted_iota(jnp.int32, (W, H), 1))
    i_half = idxs.reshape(G, W2, H)
    k_half = keys.reshape(G, W2, H)
    for b in range(G):
        t_cos = k_half[b, W2 - 1, H - 1]
        t_idx = i_half[b, W2 - 1, H - 1]
        cb = cos_ref[b]
        mask = (cb < t_cos) | ((cb == t_cos) & (p2d <= t_idx))
        mask_ref[b, 0] = (mask.astype(jnp.float32) - cb) + cb
        idx_ref[b] = i_half[b]


def kernel(x, y):
    cos, raw = pl.pallas_call(
        _dense_body,
        grid=(B,),
        in_specs=[
            pl.BlockSpec((1, C, W, H), lambda b: (b, 0, 0, 0)),
            pl.BlockSpec((1, C, W, H), lambda b: (b, 0, 0, 0)),
        ],
        out_specs=[
            pl.BlockSpec((1, W, H), lambda b: (b, 0, 0)),
            pl.BlockSpec((1, 1, W, H), lambda b: (b, 0, 0, 0)),
        ],
        out_shape=[
            jax.ShapeDtypeStruct((B, W, H), jnp.float32),
            jax.ShapeDtypeStruct((B, 1, W, H), jnp.float32),
        ],
    )(x, y)
    mask, idx = pl.pallas_call(
        _sort_body,
        grid=(B // G,),
        in_specs=[pl.BlockSpec((G, W, H), lambda g: (g, 0, 0))],
        out_specs=[
            pl.BlockSpec((G, 1, W, H), lambda g: (g, 0, 0, 0)),
            pl.BlockSpec((G, K // H, H), lambda g: (g, 0, 0)),
        ],
        out_shape=[
            jax.ShapeDtypeStruct((B, 1, W, H), jnp.float32),
            jax.ShapeDtypeStruct((B, K // H, H), jnp.int32),
        ],
    )(cos)
    return mask, raw, idx.reshape(B, K)
